# baseline (speedup 1.0000x reference)
import functools

import numpy as np
import jax
import jax.numpy as jnp
from jax import lax
from jax.experimental import pallas as pl
from jax.experimental.pallas import tpu as pltpu
from jax.experimental.pallas import tpu_sc as plsc

N = 10000
DEG = 16
E = N * DEG
C = 4
B = 32
R = B * C
L = 16
G = 2
BC = B // G
RC = R // G
RPT = RC // 32

_PI = (7919 * np.arange(N, dtype=np.int64) + 3) % N
_MINV = np.zeros(N, dtype=np.int32)
_MINV[_PI] = np.arange(N, dtype=np.int32)


def _wid():
    return lax.axis_index("s") * 2 + lax.axis_index("c")


def _sc_perm(hm_hbm, idx_hbm, hn_hbm,
             idx_v, r0_v, r1_v, gout_v, sem_i, sem0, sem1):
    wid = _wid()
    rows_v = (r0_v, r1_v)
    sems = (sem0, sem1)
    cpi = pltpu.async_copy(idx_hbm, idx_v, sem_i)
    cps = [
        pltpu.async_copy(hm_hbm.at[wid * RPT + r], rows_v[r], sems[r])
        for r in range(RPT)
    ]
    cpi.wait()
    for r in range(RPT):
        cps[r].wait()
        row_v = rows_v[r]

        def body(j, _, row_v=row_v):
            idx16 = idx_v[pl.ds(j * L, L)]
            gout_v[pl.ds(j * L, L)] = plsc.load_gather(row_v, [idx16])
            return 0

        lax.fori_loop(0, N // L, body, 0, unroll=8)
        pltpu.sync_copy(gout_v, hn_hbm.at[wid * RPT + r])


def _lin1_body(x_ref, w1_ref, out_ref):
    x = x_ref[0]
    hs = [jnp.sum(x * w1_ref[c], axis=0, keepdims=True) for c in range(C)]
    out_ref[0] = jnp.concatenate(hs, axis=0)


def _lin3_body(h_ref, b1_ref, gam_ref, bet_ref, w3_ref, b3_ref, x_ref,
               out_ref):
    h = h_ref[0] + b1_ref[...]
    mu = jnp.mean(h, axis=0, keepdims=True)
    d = h - mu
    var = jnp.mean(d * d, axis=0, keepdims=True)
    hn = d * lax.rsqrt(var + 1e-5) * gam_ref[...] + bet_ref[...]
    hn = jnp.where(hn > 0, hn, jnp.exp(jnp.minimum(hn, 0.0)) - 1.0)
    acc = x_ref[0] + b3_ref[...]
    for c in range(C):
        acc = acc + hn[c][None, :] * w3_ref[c]
    out_ref[0] = acc


_SC_MESH = plsc.VectorSubcoreMesh(core_axis_name="c", subcore_axis_name="s")
_SC_PARAMS = pltpu.CompilerParams(needs_layout_passes=False)


def kernel(x, w1_vals, b1, w3_vals, b3, gamma, beta, rows1, cols1, rows3, cols3):
    minv = jnp.asarray(_MINV)
    xr = x.reshape(B, DEG, N)
    b3r = b3.reshape(DEG, N)

    w1t = w1_vals.reshape(DEG, N, C).transpose(2, 0, 1)
    w3t = w3_vals.reshape(DEG, N, C).transpose(2, 0, 1)
    b1t = b1.reshape(N, C).T
    gamt = gamma.reshape(N, C).T
    bett = beta.reshape(N, C).T

    perm = functools.partial(
        pl.kernel,
        out_type=jax.ShapeDtypeStruct((RC, N), jnp.float32),
        mesh=_SC_MESH,
        scratch_types=[
            pltpu.VMEM((N,), jnp.int32),
            pltpu.VMEM((N,), jnp.float32),
            pltpu.VMEM((N,), jnp.float32),
            pltpu.VMEM((N,), jnp.float32),
            pltpu.SemaphoreType.DMA,
            pltpu.SemaphoreType.DMA,
            pltpu.SemaphoreType.DMA,
        ],
        compiler_params=_SC_PARAMS,
    )(_sc_perm)

    out = None
    for g in range(G):
        hm_g = pl.pallas_call(
            _lin1_body,
            grid=(BC,),
            in_specs=[
                pl.BlockSpec((1, DEG, N), lambda b, g=g: (g * BC + b, 0, 0)),
                pl.BlockSpec((C, DEG, N), lambda b: (0, 0, 0)),
            ],
            out_specs=pl.BlockSpec((1, C, N), lambda b: (b, 0, 0)),
            out_shape=jax.ShapeDtypeStruct((BC, C, N), jnp.float32),
        )(xr, w1t)
        hn_g = perm(hm_g.reshape(RC, N), minv).reshape(BC, C, N)
        in_specs = [
            pl.BlockSpec((1, C, N), lambda b: (b, 0, 0)),
            pl.BlockSpec((C, N), lambda b: (0, 0)),
            pl.BlockSpec((C, N), lambda b: (0, 0)),
            pl.BlockSpec((C, N), lambda b: (0, 0)),
            pl.BlockSpec((C, DEG, N), lambda b: (0, 0, 0)),
            pl.BlockSpec((DEG, N), lambda b: (0, 0)),
            pl.BlockSpec((1, DEG, N), lambda b, g=g: (g * BC + b, 0, 0)),
        ]
        args = [hn_g, b1t, gamt, bett, w3t, b3r, xr]
        body = _lin3_body
        aliases = {}
        if out is not None:
            in_specs.append(
                pl.BlockSpec((1, DEG, N), lambda b, g=g: (g * BC + b, 0, 0)))
            args.append(out)
            body = lambda h, b1r, ga, be, w3, b3_, x_, _acc, o: _lin3_body(
                h, b1r, ga, be, w3, b3_, x_, o)
            aliases = {7: 0}
        out = pl.pallas_call(
            body,
            grid=(BC,),
            in_specs=in_specs,
            out_specs=pl.BlockSpec((1, DEG, N), lambda b, g=g: (g * BC + b, 0, 0)),
            out_shape=jax.ShapeDtypeStruct((B, DEG, N), jnp.float32),
            input_output_aliases=aliases,
        )(*args)

    return out.reshape(B, E)


# speedup vs baseline: 2.1934x; 2.1934x over previous
import functools

import numpy as np
import jax
import jax.numpy as jnp
from jax import lax
from jax.experimental import pallas as pl
from jax.experimental.pallas import tpu as pltpu
from jax.experimental.pallas import tpu_sc as plsc

N = 10000
DEG = 16
E = N * DEG
C = 4
B = 32
R = B * C
L = 16
G = 4
BC = B // G
RC = R // G
RPT = RC // 32

_PI = (7919 * np.arange(N, dtype=np.int64) + 3) % N
_MINV = np.zeros(N, dtype=np.int32)
_MINV[_PI] = np.arange(N, dtype=np.int32)


def _deinterleave_row(slab_v, out_v, c):
    lane = lax.iota(jnp.int32, L)

    def body(j, _):
        idx16 = (j * L + lane) * C + c
        out_v[pl.ds(j * L, L)] = plsc.load_gather(slab_v, [idx16])
        return 0

    lax.fori_loop(0, N // L, body, 0, unroll=8)


def _wid():
    return lax.axis_index("s") * 2 + lax.axis_index("c")


def _sc_prep_w1(w1_hbm, w1t_hbm, slab_v, out_v):
    wid = _wid()
    k = wid % DEG
    h = wid // DEG
    pltpu.sync_copy(w1_hbm.at[k], slab_v)
    for ci in range(2):
        c = 2 * h + ci
        _deinterleave_row(slab_v, out_v, c)
        pltpu.sync_copy(out_v, w1t_hbm.at[c, k])


def _sc_prep_rest(w3_hbm, b1_hbm, gam_hbm, bet_hbm,
                  w3t_hbm, b1t_hbm, gamt_hbm, bett_hbm, slab_v, out_v):
    wid = _wid()
    k = wid % DEG
    h = wid // DEG
    pltpu.sync_copy(w3_hbm.at[k], slab_v)
    for ci in range(2):
        c = 2 * h + ci
        _deinterleave_row(slab_v, out_v, c)
        pltpu.sync_copy(out_v, w3t_hbm.at[c, k])
    a = wid // C
    c4 = wid % C
    for ai, (src_hbm, dst_hbm) in enumerate(
        ((b1_hbm, b1t_hbm), (gam_hbm, gamt_hbm), (bet_hbm, bett_hbm))
    ):
        @pl.when(a == ai)
        def _(src_hbm=src_hbm, dst_hbm=dst_hbm):
            pltpu.sync_copy(src_hbm, slab_v)
            _deinterleave_row(slab_v, out_v, c4)
            pltpu.sync_copy(out_v, dst_hbm.at[c4])


def _sc_perm(hm_hbm, idx_hbm, hn_hbm,
             idx_v, r0_v, r1_v, gout_v, sem_i, sem0, sem1):
    wid = _wid()
    rows_v = (r0_v, r1_v)
    sems = (sem0, sem1)
    cpi = pltpu.async_copy(idx_hbm, idx_v, sem_i)
    cps = [
        pltpu.async_copy(hm_hbm.at[wid * RPT + r], rows_v[r], sems[r])
        for r in range(RPT)
    ]
    cpi.wait()
    for r in range(RPT):
        cps[r].wait()
        row_v = rows_v[r]

        def body(j, _, row_v=row_v):
            idx16 = idx_v[pl.ds(j * L, L)]
            gout_v[pl.ds(j * L, L)] = plsc.load_gather(row_v, [idx16])
            return 0

        lax.fori_loop(0, N // L, body, 0, unroll=8)
        pltpu.sync_copy(gout_v, hn_hbm.at[wid * RPT + r])


def _lin1_body(x_ref, w1_ref, out_ref):
    x = x_ref[0]
    hs = [jnp.sum(x * w1_ref[c], axis=0, keepdims=True) for c in range(C)]
    out_ref[0] = jnp.concatenate(hs, axis=0)


def _lin3_body(h_ref, b1_ref, gam_ref, bet_ref, w3_ref, b3_ref, x_ref,
               out_ref):
    h = h_ref[0] + b1_ref[...]
    mu = jnp.mean(h, axis=0, keepdims=True)
    d = h - mu
    var = jnp.mean(d * d, axis=0, keepdims=True)
    hn = d * lax.rsqrt(var + 1e-5) * gam_ref[...] + bet_ref[...]
    hn = jnp.where(hn > 0, hn, jnp.exp(jnp.minimum(hn, 0.0)) - 1.0)
    acc = x_ref[0] + b3_ref[...]
    for c in range(C):
        acc = acc + hn[c][None, :] * w3_ref[c]
    out_ref[0] = acc


_SC_MESH = plsc.VectorSubcoreMesh(core_axis_name="c", subcore_axis_name="s")
_SC_PARAMS = pltpu.CompilerParams(needs_layout_passes=False)


def kernel(x, w1_vals, b1, w3_vals, b3, gamma, beta, rows1, cols1, rows3, cols3):
    minv = jnp.asarray(_MINV)
    xr = x.reshape(B, DEG, N)
    b3r = b3.reshape(DEG, N)

    prep_w1 = functools.partial(
        pl.kernel,
        out_type=jax.ShapeDtypeStruct((C, DEG, N), jnp.float32),
        mesh=_SC_MESH,
        scratch_types=[
            pltpu.VMEM((N * C,), jnp.float32),
            pltpu.VMEM((N,), jnp.float32),
        ],
        compiler_params=_SC_PARAMS,
    )(_sc_prep_w1)
    w1t = prep_w1(w1_vals.reshape(DEG, N * C))

    cn = jax.ShapeDtypeStruct((C, N), jnp.float32)
    prep_rest = functools.partial(
        pl.kernel,
        out_type=(jax.ShapeDtypeStruct((C, DEG, N), jnp.float32), cn, cn, cn),
        mesh=_SC_MESH,
        scratch_types=[
            pltpu.VMEM((N * C,), jnp.float32),
            pltpu.VMEM((N,), jnp.float32),
        ],
        compiler_params=_SC_PARAMS,
    )(_sc_prep_rest)
    w3t, b1t, gamt, bett = prep_rest(
        w3_vals.reshape(DEG, N * C), b1, gamma, beta)

    perm = functools.partial(
        pl.kernel,
        out_type=jax.ShapeDtypeStruct((RC, N), jnp.float32),
        mesh=_SC_MESH,
        scratch_types=[
            pltpu.VMEM((N,), jnp.int32),
            pltpu.VMEM((N,), jnp.float32),
            pltpu.VMEM((N,), jnp.float32),
            pltpu.VMEM((N,), jnp.float32),
            pltpu.SemaphoreType.DMA,
            pltpu.SemaphoreType.DMA,
            pltpu.SemaphoreType.DMA,
        ],
        compiler_params=_SC_PARAMS,
    )(_sc_perm)

    out = None
    for g in range(G):
        hm_g = pl.pallas_call(
            _lin1_body,
            grid=(BC,),
            in_specs=[
                pl.BlockSpec((1, DEG, N), lambda b, g=g: (g * BC + b, 0, 0)),
                pl.BlockSpec((C, DEG, N), lambda b: (0, 0, 0)),
            ],
            out_specs=pl.BlockSpec((1, C, N), lambda b: (b, 0, 0)),
            out_shape=jax.ShapeDtypeStruct((BC, C, N), jnp.float32),
        )(xr, w1t)
        hn_g = perm(hm_g.reshape(RC, N), minv).reshape(BC, C, N)
        in_specs = [
            pl.BlockSpec((1, C, N), lambda b: (b, 0, 0)),
            pl.BlockSpec((C, N), lambda b: (0, 0)),
            pl.BlockSpec((C, N), lambda b: (0, 0)),
            pl.BlockSpec((C, N), lambda b: (0, 0)),
            pl.BlockSpec((C, DEG, N), lambda b: (0, 0, 0)),
            pl.BlockSpec((DEG, N), lambda b: (0, 0)),
            pl.BlockSpec((1, DEG, N), lambda b, g=g: (g * BC + b, 0, 0)),
        ]
        args = [hn_g, b1t, gamt, bett, w3t, b3r, xr]
        body = _lin3_body
        aliases = {}
        if out is not None:
            in_specs.append(
                pl.BlockSpec((1, DEG, N), lambda b, g=g: (g * BC + b, 0, 0)))
            args.append(out)
            body = lambda h, b1r, ga, be, w3, b3_, x_, _acc, o: _lin3_body(
                h, b1r, ga, be, w3, b3_, x_, o)
            aliases = {7: 0}
        out = pl.pallas_call(
            body,
            grid=(BC,),
            in_specs=in_specs,
            out_specs=pl.BlockSpec((1, DEG, N), lambda b, g=g: (g * BC + b, 0, 0)),
            out_shape=jax.ShapeDtypeStruct((B, DEG, N), jnp.float32),
            input_output_aliases=aliases,
        )(*args)

    return out.reshape(B, E)


# speedup vs baseline: 2.4170x; 1.1019x over previous
import functools

import numpy as np
import jax
import jax.numpy as jnp
from jax import lax
from jax.experimental import pallas as pl
from jax.experimental.pallas import tpu as pltpu
from jax.experimental.pallas import tpu_sc as plsc

N = 10000
DEG = 16
E = N * DEG
C = 4
B = 32
R = B * C
L = 16
G = 2
BC = B // G
RC = R // G
RPT = RC // 32

_PI = (7919 * np.arange(N, dtype=np.int64) + 3) % N
_MINV = np.zeros(N, dtype=np.int32)
_MINV[_PI] = np.arange(N, dtype=np.int32)


def _deinterleave_row(slab_v, out_v, c):
    lane = lax.iota(jnp.int32, L)

    def body(j, _):
        idx16 = (j * L + lane) * C + c
        out_v[pl.ds(j * L, L)] = plsc.load_gather(slab_v, [idx16])
        return 0

    lax.fori_loop(0, N // L, body, 0, unroll=8)


def _wid():
    return lax.axis_index("s") * 2 + lax.axis_index("c")


def _sc_prep_w1(w1_hbm, w1t_hbm, slab_v, out_v):
    wid = _wid()
    k = wid % DEG
    h = wid // DEG
    pltpu.sync_copy(w1_hbm.at[k], slab_v)
    for ci in range(2):
        c = 2 * h + ci
        _deinterleave_row(slab_v, out_v, c)
        pltpu.sync_copy(out_v, w1t_hbm.at[c, k])


def _sc_prep_rest(w3_hbm, b1_hbm, gam_hbm, bet_hbm,
                  w3t_hbm, b1t_hbm, gamt_hbm, bett_hbm, slab_v, out_v):
    wid = _wid()
    k = wid % DEG
    h = wid // DEG
    pltpu.sync_copy(w3_hbm.at[k], slab_v)
    for ci in range(2):
        c = 2 * h + ci
        _deinterleave_row(slab_v, out_v, c)
        pltpu.sync_copy(out_v, w3t_hbm.at[c, k])
    a = wid // C
    c4 = wid % C
    for ai, (src_hbm, dst_hbm) in enumerate(
        ((b1_hbm, b1t_hbm), (gam_hbm, gamt_hbm), (bet_hbm, bett_hbm))
    ):
        @pl.when(a == ai)
        def _(src_hbm=src_hbm, dst_hbm=dst_hbm):
            pltpu.sync_copy(src_hbm, slab_v)
            _deinterleave_row(slab_v, out_v, c4)
            pltpu.sync_copy(out_v, dst_hbm.at[c4])


def _sc_perm(hm_hbm, idx_hbm, hn_hbm,
             idx_v, r0_v, r1_v, g0_v, g1_v, sem_i, sem0, sem1, semo0, semo1):
    wid = _wid()
    rows_v = (r0_v, r1_v)
    gouts_v = (g0_v, g1_v)
    sems = (sem0, sem1)
    osems = (semo0, semo1)
    cpi = pltpu.async_copy(idx_hbm, idx_v, sem_i)
    cps = [
        pltpu.async_copy(hm_hbm.at[wid * RPT + r], rows_v[r], sems[r])
        for r in range(RPT)
    ]
    cpi.wait()
    stores = []
    for r in range(RPT):
        cps[r].wait()
        row_v = rows_v[r]
        gout_v = gouts_v[r]

        def body(j, _, row_v=row_v, gout_v=gout_v):
            idx16 = idx_v[pl.ds(j * L, L)]
            gout_v[pl.ds(j * L, L)] = plsc.load_gather(row_v, [idx16])
            return 0

        lax.fori_loop(0, N // L, body, 0, unroll=8)
        stores.append(
            pltpu.async_copy(gout_v, hn_hbm.at[wid * RPT + r], osems[r]))
    for s in stores:
        s.wait()


def _lin1_body(x_ref, w1_ref, out_ref):
    x = x_ref[0]
    hs = [jnp.sum(x * w1_ref[c], axis=0, keepdims=True) for c in range(C)]
    out_ref[0] = jnp.concatenate(hs, axis=0)


def _lin3_body(h_ref, b1_ref, gam_ref, bet_ref, w3_ref, b3_ref, x_ref,
               out_ref):
    h = h_ref[0] + b1_ref[...]
    mu = jnp.mean(h, axis=0, keepdims=True)
    d = h - mu
    var = jnp.mean(d * d, axis=0, keepdims=True)
    hn = d * lax.rsqrt(var + 1e-5) * gam_ref[...] + bet_ref[...]
    hn = jnp.where(hn > 0, hn, jnp.exp(jnp.minimum(hn, 0.0)) - 1.0)
    acc = x_ref[0] + b3_ref[...]
    for c in range(C):
        acc = acc + hn[c][None, :] * w3_ref[c]
    out_ref[0] = acc


_SC_MESH = plsc.VectorSubcoreMesh(core_axis_name="c", subcore_axis_name="s")
_SC_PARAMS = pltpu.CompilerParams(needs_layout_passes=False)


def kernel(x, w1_vals, b1, w3_vals, b3, gamma, beta, rows1, cols1, rows3, cols3):
    minv = jnp.asarray(_MINV)
    xr = x.reshape(B, DEG, N)
    b3r = b3.reshape(DEG, N)

    prep_w1 = functools.partial(
        pl.kernel,
        out_type=jax.ShapeDtypeStruct((C, DEG, N), jnp.float32),
        mesh=_SC_MESH,
        scratch_types=[
            pltpu.VMEM((N * C,), jnp.float32),
            pltpu.VMEM((N,), jnp.float32),
        ],
        compiler_params=_SC_PARAMS,
    )(_sc_prep_w1)
    w1t = prep_w1(w1_vals.reshape(DEG, N * C))

    cn = jax.ShapeDtypeStruct((C, N), jnp.float32)
    prep_rest = functools.partial(
        pl.kernel,
        out_type=(jax.ShapeDtypeStruct((C, DEG, N), jnp.float32), cn, cn, cn),
        mesh=_SC_MESH,
        scratch_types=[
            pltpu.VMEM((N * C,), jnp.float32),
            pltpu.VMEM((N,), jnp.float32),
        ],
        compiler_params=_SC_PARAMS,
    )(_sc_prep_rest)
    w3t, b1t, gamt, bett = prep_rest(
        w3_vals.reshape(DEG, N * C), b1, gamma, beta)

    perm = functools.partial(
        pl.kernel,
        out_type=jax.ShapeDtypeStruct((RC, N), jnp.float32),
        mesh=_SC_MESH,
        scratch_types=[
            pltpu.VMEM((N,), jnp.int32),
            pltpu.VMEM((N,), jnp.float32),
            pltpu.VMEM((N,), jnp.float32),
            pltpu.VMEM((N,), jnp.float32),
            pltpu.VMEM((N,), jnp.float32),
            pltpu.SemaphoreType.DMA,
            pltpu.SemaphoreType.DMA,
            pltpu.SemaphoreType.DMA,
            pltpu.SemaphoreType.DMA,
            pltpu.SemaphoreType.DMA,
        ],
        compiler_params=_SC_PARAMS,
    )(_sc_perm)

    out = None
    for g in range(G):
        hm_g = pl.pallas_call(
            _lin1_body,
            grid=(BC,),
            in_specs=[
                pl.BlockSpec((1, DEG, N), lambda b, g=g: (g * BC + b, 0, 0)),
                pl.BlockSpec((C, DEG, N), lambda b: (0, 0, 0)),
            ],
            out_specs=pl.BlockSpec((1, C, N), lambda b: (b, 0, 0)),
            out_shape=jax.ShapeDtypeStruct((BC, C, N), jnp.float32),
        )(xr, w1t)
        hn_g = perm(hm_g.reshape(RC, N), minv).reshape(BC, C, N)
        in_specs = [
            pl.BlockSpec((1, C, N), lambda b: (b, 0, 0)),
            pl.BlockSpec((C, N), lambda b: (0, 0)),
            pl.BlockSpec((C, N), lambda b: (0, 0)),
            pl.BlockSpec((C, N), lambda b: (0, 0)),
            pl.BlockSpec((C, DEG, N), lambda b: (0, 0, 0)),
            pl.BlockSpec((DEG, N), lambda b: (0, 0)),
            pl.BlockSpec((1, DEG, N), lambda b, g=g: (g * BC + b, 0, 0)),
        ]
        args = [hn_g, b1t, gamt, bett, w3t, b3r, xr]
        body = _lin3_body
        aliases = {}
        if out is not None:
            in_specs.append(
                pl.BlockSpec((1, DEG, N), lambda b, g=g: (g * BC + b, 0, 0)))
            args.append(out)
            body = lambda h, b1r, ga, be, w3, b3_, x_, _acc, o: _lin3_body(
                h, b1r, ga, be, w3, b3_, x_, o)
            aliases = {7: 0}
        out = pl.pallas_call(
            body,
            grid=(BC,),
            in_specs=in_specs,
            out_specs=pl.BlockSpec((1, DEG, N), lambda b, g=g: (g * BC + b, 0, 0)),
            out_shape=jax.ShapeDtypeStruct((B, DEG, N), jnp.float32),
            input_output_aliases=aliases,
        )(*args)

    return out.reshape(B, E)
